# baseline (speedup 1.0000x reference)
import jax
import jax.numpy as jnp
from jax import lax
from jax.experimental import pallas as pl
from jax.experimental.pallas import tpu as pltpu
from jax.experimental.pallas import tpu_sc as plsc

BATCH = 4096
MAXLEN = 200
EMBED = 64

NC = 2
NS = 16
NW = NC * NS

BW = BATCH // NW
L = 16
HALF = 32
TPAD = 132
NBUF = 4


def _body(xt_hbm, table_hbm, pos_hbm, out_hbm,
          pos_v, idxr_v, idx2_v, g_v, t_v, gsem, osem):
  wid = lax.axis_index("s") * NC + lax.axis_index("c")
  b0 = wid * BW

  pltpu.sync_copy(pos_hbm, pos_v)

  iota = lax.iota(jnp.int32, L)
  rows_c = [v * L + iota for v in range(EMBED // L)]

  def build_idx(l, b):
    pltpu.sync_copy(xt_hbm.at[l, pl.ds(b0, BW)], idxr_v)
    idx2 = idx2_v[b]
    for v in range(BW // L):
      iv = idxr_v[pl.ds(v * L, L)] * 2
      pos2 = (v * L + iota) * 2
      plsc.store_scatter(idx2, [pos2], iv)
      plsc.store_scatter(idx2, [pos2 + 1], iv + 1)

  def start_gather(b):
    pltpu.async_copy(table_hbm.at[idx2_v[b].at[pl.ds(0, BW)]],
                     g_v[b].at[pl.ds(0, BW)], gsem[b])
    pltpu.async_copy(table_hbm.at[idx2_v[b].at[pl.ds(BW, BW)]],
                     g_v[b].at[pl.ds(BW, BW)], gsem[b])

  def wait_gather(b):
    pltpu.make_async_copy(table_hbm.at[idx2_v[b].at[pl.ds(0, BW)]],
                          g_v[b].at[pl.ds(0, BW)], gsem[b]).wait()
    pltpu.make_async_copy(table_hbm.at[idx2_v[b].at[pl.ds(BW, BW)]],
                          g_v[b].at[pl.ds(BW, BW)], gsem[b]).wait()

  def corner_turn(l, b):
    g, t = g_v[b], t_v[b]
    pos_l = [pos_v[l, pl.ds(v * L, L)] for v in range(EMBED // L)]
    te_c = [r // 8 for r in rows_c]
    ee_c = [r % 8 for r in rows_c]

    @pl.loop(0, BW, unroll=8)
    def _(j):
      colj = jnp.full((L,), j, jnp.int32)
      for v in range(EMBED // L):
        val = g[2 * j + v // 2, pl.ds((v % 2) * L, L)] + pos_l[v]
        plsc.store_scatter(t, [te_c[v], ee_c[v], colj], val)

  def start_out(l, b):
    pltpu.async_copy(t_v[b].at[:, :, pl.ds(0, BW)],
                     out_hbm.at[l, :, wid, :, :], osem[b])

  def wait_out(l, b):
    pltpu.make_async_copy(t_v[b].at[:, :, pl.ds(0, BW)],
                          out_hbm.at[l, :, wid, :, :], osem[b]).wait()

  def stage(l, b):
    wait_out(l - NBUF, b)
    build_idx(l, b)
    start_gather(b)
    p = (b + 1) % NBUF
    wait_gather(p)
    corner_turn(l - (NBUF - 1), p)
    start_out(l - (NBUF - 1), p)

  for l in range(NBUF):
    build_idx(l, l)
    start_gather(l)
  wait_gather(0)
  corner_turn(0, 0)
  start_out(0, 0)

  @pl.loop(NBUF, MAXLEN, step=NBUF)
  def _(l):
    for j in range(NBUF):
      stage(l + j, j)

  for m in range(MAXLEN - NBUF + 1, MAXLEN):
    b = m % NBUF
    wait_gather(b)
    corner_turn(m, b)
    start_out(m, b)
  for m in range(MAXLEN - NBUF, MAXLEN):
    wait_out(m, m % NBUF)


@jax.jit
def _embed(xt, table2, pos_table):
  mesh = plsc.VectorSubcoreMesh(
      core_axis_name="c", subcore_axis_name="s", num_cores=NC, num_subcores=NS)
  k = pl.kernel(
      _body,
      out_type=jax.ShapeDtypeStruct((MAXLEN, 8, BATCH // 128, 8, 128),
                                    jnp.float32),
      mesh=mesh,
      compiler_params=pltpu.CompilerParams(
          use_tc_tiling_on_sc=False, needs_layout_passes=False),
      scratch_types=[
          pltpu.VMEM((MAXLEN, EMBED), jnp.float32),
          pltpu.VMEM((BW,), jnp.int32),
          [pltpu.VMEM((2 * BW,), jnp.int32) for _ in range(NBUF)],
          [pltpu.VMEM((2 * BW, HALF), jnp.float32) for _ in range(NBUF)],
          [pltpu.VMEM((8, 8, TPAD), jnp.float32) for _ in range(NBUF)],
          [pltpu.SemaphoreType.DMA for _ in range(NBUF)],
          [pltpu.SemaphoreType.DMA for _ in range(NBUF)],
      ],
  )
  return k(xt, table2, pos_table)


def kernel(x, token_table, pos_table):
  xt = x.astype(jnp.int32).T
  table2 = token_table.reshape(2 * token_table.shape[0], HALF)
  out5 = _embed(xt, table2, pos_table)
  return jnp.transpose(out5, (2, 4, 0, 1, 3)).reshape(BATCH, MAXLEN, EMBED)


# speedup vs baseline: 1.2335x; 1.2335x over previous
import jax
import jax.numpy as jnp
from jax import lax
from jax.experimental import pallas as pl
from jax.experimental.pallas import tpu as pltpu
from jax.experimental.pallas import tpu_sc as plsc

BATCH = 4096
MAXLEN = 200
EMBED = 64

NC = 2
NS = 16
NW = NC * NS

BW = BATCH // NW
L = 16
HALF = 32
TPAD = 132
NBUF = 4


def _body(xt_hbm, table_hbm, pos_hbm, out_hbm,
          pos_v, idxr_v, idx2_v, g_v, t_v, gsem, osem):
  wid = lax.axis_index("s") * NC + lax.axis_index("c")
  b0 = wid * BW

  pltpu.sync_copy(pos_hbm, pos_v)

  iota = lax.iota(jnp.int32, L)
  rows_c = [v * L + iota for v in range(EMBED // L)]

  def build_idx(l, b):
    pltpu.sync_copy(xt_hbm.at[l, pl.ds(b0, BW)], idxr_v)
    idx2 = idx2_v[b]
    for v in range(BW // L):
      iv = idxr_v[pl.ds(v * L, L)] * 2
      pos2 = (v * L + iota) * 2
      plsc.store_scatter(idx2, [pos2], iv)
      plsc.store_scatter(idx2, [pos2 + 1], iv + 1)

  def start_gather(b):
    pltpu.async_copy(table_hbm.at[idx2_v[b].at[pl.ds(0, BW)]],
                     g_v[b].at[pl.ds(0, BW)], gsem[b])
    pltpu.async_copy(table_hbm.at[idx2_v[b].at[pl.ds(BW, BW)]],
                     g_v[b].at[pl.ds(BW, BW)], gsem[b])

  def wait_gather(b):
    pltpu.make_async_copy(table_hbm.at[idx2_v[b].at[pl.ds(0, BW)]],
                          g_v[b].at[pl.ds(0, BW)], gsem[b]).wait()
    pltpu.make_async_copy(table_hbm.at[idx2_v[b].at[pl.ds(BW, BW)]],
                          g_v[b].at[pl.ds(BW, BW)], gsem[b]).wait()

  def corner_turn(l, b):
    g, t = g_v[b], t_v[b]
    pos_l = [pos_v[l, pl.ds(v * L, L)] for v in range(EMBED // L)]
    te_c = [r // 8 for r in rows_c]
    ee_c = [r % 8 for r in rows_c]

    JB = 4
    NV = EMBED // L

    @pl.loop(0, BW, step=JB)
    def _(j0):
      js = [j0 + u for u in range(JB)]
      cols = [jnp.full((L,), j, jnp.int32) for j in js]
      vals = [[g[2 * j + v // 2, pl.ds((v % 2) * L, L)] for v in range(NV)]
              for j in js]
      sums = [[vals[u][v] + pos_l[v] for v in range(NV)] for u in range(JB)]
      for u in range(JB):
        for v in range(NV):
          plsc.store_scatter(t, [te_c[v], ee_c[v], cols[u]], sums[u][v])

  def start_out(l, b):
    pltpu.async_copy(t_v[b].at[:, :, pl.ds(0, BW)],
                     out_hbm.at[l, :, wid, :, :], osem[b])

  def wait_out(l, b):
    pltpu.make_async_copy(t_v[b].at[:, :, pl.ds(0, BW)],
                          out_hbm.at[l, :, wid, :, :], osem[b]).wait()

  def stage(l, b):
    wait_out(l - NBUF, b)
    build_idx(l, b)
    start_gather(b)
    p = (b + 1) % NBUF
    wait_gather(p)
    corner_turn(l - (NBUF - 1), p)
    start_out(l - (NBUF - 1), p)

  for l in range(NBUF):
    build_idx(l, l)
    start_gather(l)
  wait_gather(0)
  corner_turn(0, 0)
  start_out(0, 0)

  @pl.loop(NBUF, MAXLEN, step=NBUF)
  def _(l):
    for j in range(NBUF):
      stage(l + j, j)

  for m in range(MAXLEN - NBUF + 1, MAXLEN):
    b = m % NBUF
    wait_gather(b)
    corner_turn(m, b)
    start_out(m, b)
  for m in range(MAXLEN - NBUF, MAXLEN):
    wait_out(m, m % NBUF)


@jax.jit
def _embed(xt, table2, pos_table):
  mesh = plsc.VectorSubcoreMesh(
      core_axis_name="c", subcore_axis_name="s", num_cores=NC, num_subcores=NS)
  k = pl.kernel(
      _body,
      out_type=jax.ShapeDtypeStruct((MAXLEN, 8, BATCH // 128, 8, 128),
                                    jnp.float32),
      mesh=mesh,
      compiler_params=pltpu.CompilerParams(
          use_tc_tiling_on_sc=False, needs_layout_passes=False),
      scratch_types=[
          pltpu.VMEM((MAXLEN, EMBED), jnp.float32),
          pltpu.VMEM((BW,), jnp.int32),
          [pltpu.VMEM((2 * BW,), jnp.int32) for _ in range(NBUF)],
          [pltpu.VMEM((2 * BW, HALF), jnp.float32) for _ in range(NBUF)],
          [pltpu.VMEM((8, 8, TPAD), jnp.float32) for _ in range(NBUF)],
          [pltpu.SemaphoreType.DMA for _ in range(NBUF)],
          [pltpu.SemaphoreType.DMA for _ in range(NBUF)],
      ],
  )
  return k(xt, table2, pos_table)


def kernel(x, token_table, pos_table):
  xt = x.astype(jnp.int32).T
  table2 = token_table.reshape(2 * token_table.shape[0], HALF)
  out5 = _embed(xt, table2, pos_table)
  return jnp.transpose(out5, (2, 4, 0, 1, 3)).reshape(BATCH, MAXLEN, EMBED)
